# baseline (speedup 1.0000x reference)
import jax
import jax.numpy as jnp
from jax.experimental import pallas as pl

N = 10000
E = 160000
D = 128
NRBF = 20
CUTOFF = 5.0
NINT = 3
HEADS = 10


def _ssp(x):
    return jnp.logaddexp(x, 0.0) - jnp.log(2.0)


def _copy_kernel(x_ref, o_ref):
    o_ref[...] = x_ref[...]


def kernel(Z, Rij, idx_i, idx_j, emb, in2f_W, fw1, fb1, fw2, fb2,
           o1, ob1, o2, ob2, gW, ga_src, ga_dst, gb):
    d_ij = jnp.linalg.norm(Rij, axis=1)
    offsets = jnp.linspace(0.0, CUTOFF, NRBF)
    coeff = -0.5 / (offsets[1] - offsets[0]) ** 2
    f_ij = jnp.exp(coeff * (d_ij[:, None] - offsets[None, :]) ** 2)
    rcut_ij = 0.5 * (jnp.cos(jnp.pi * d_ij / CUTOFF) + 1.0) * (d_ij < CUTOFF).astype(jnp.float32)
    x = emb[Z]
    loop = jnp.arange(N)
    src2 = jnp.concatenate([idx_i, loop])
    dst2 = jnp.concatenate([idx_j, loop])
    for t in range(NINT):
        h = (x @ gW[t]).reshape(N, HEADS, D)
        a_s = jnp.sum(h * ga_src[t][None, :, :], axis=-1)
        a_d = jnp.sum(h * ga_dst[t][None, :, :], axis=-1)
        e = a_s[src2] + a_d[dst2]
        e = jnp.where(e > 0, e, 0.2 * e)
        emax = jax.ops.segment_max(e, dst2, num_segments=N)
        a = jnp.exp(e - emax[dst2])
        denom = jax.ops.segment_sum(a, dst2, num_segments=N)
        a = a / (denom[dst2] + 1e-16)
        gat_out = jax.ops.segment_sum(h[src2] * a[:, :, None], dst2, num_segments=N)
        a_out = gat_out.mean(axis=1) + gb[t]
        xf = x @ in2f_W[t]
        Wij = _ssp(f_ij @ fw1[t] + fb1[t]) @ fw2[t] + fb2[t]
        Wij = Wij * rcut_ij[:, None]
        x_ij = xf[idx_j] * Wij
        agg = jax.ops.segment_sum(x_ij, idx_i, num_segments=N)
        v = _ssp(agg @ o1[t] + ob1[t]) @ o2[t] + ob2[t]
        x = x + v + a_out

    x = pl.pallas_call(
        _copy_kernel,
        out_shape=jax.ShapeDtypeStruct(x.shape, x.dtype),
    )(x)
    return x


# speedup vs baseline: 3.8868x; 3.8868x over previous
import functools
import jax
import jax.numpy as jnp
from jax.experimental import pallas as pl
from jax.experimental.pallas import tpu as pltpu

N = 10000
E = 160000
D = 128
NRBF = 20
CUTOFF = 5.0
NINT = 3
HEADS = 10
E2 = E + N

BN = 512
BE = 512

NP_ = 10240
E2P = 170496
EP = 160256

_DELTA = CUTOFF / (NRBF - 1)
_COEFF = -0.5 / _DELTA ** 2


def _ssp(x):
    return jnp.logaddexp(x, 0.0) - jnp.log(2.0)


def _full(shape):
    return pl.BlockSpec(shape, lambda i: (0,) * len(shape))


def _rows(bs, width):
    return pl.BlockSpec((bs, width), lambda i: (i, 0))


def _dense1_body(x_ref, gw_ref, i2f_ref, as_ref, ad_ref,
                 h_ref, xf_ref, sa_ref, sd_ref):
    xb = x_ref[...]
    h_ref[...] = jnp.dot(xb, gw_ref[...], preferred_element_type=jnp.float32)
    xf_ref[...] = jnp.dot(xb, i2f_ref[...], preferred_element_type=jnp.float32)
    sa_ref[...] = jnp.dot(xb, as_ref[...], preferred_element_type=jnp.float32)
    sd_ref[...] = jnp.dot(xb, ad_ref[...], preferred_element_type=jnp.float32)


def _attnw_body(gs_ref, gd_ref, w_ref):
    e = gs_ref[...] + gd_ref[...]
    e = jnp.where(e > 0, e, 0.2 * e)
    w_ref[...] = jnp.exp(e)


def _edgea_body(w_ref, gden_ref, a_ref):
    a_ref[...] = w_ref[...] / (gden_ref[...] + 1e-16)


def _contrib_body(a_ref, gh_ref, c_ref):
    a = a_ref[...]
    g = gh_ref[...]
    acc = g[:, 0:D] * a[:, 0:1]
    for hd in range(1, HEADS):
        acc += g[:, hd * D:(hd + 1) * D] * a[:, hd:hd + 1]
    c_ref[...] = acc


def _frc_body(r_ref, frc_ref):
    r = r_ref[...]
    d2 = jnp.sum(r * r, axis=1, keepdims=True)
    d = jnp.sqrt(d2)
    lane = jax.lax.broadcasted_iota(jnp.int32, r.shape, 1)
    offs = lane.astype(jnp.float32) * _DELTA
    f = jnp.exp(_COEFF * (d - offs) ** 2)
    rcut = 0.5 * (jnp.cos(jnp.pi * d / CUTOFF) + 1.0)
    rcut = rcut * (d < CUTOFF).astype(jnp.float32)
    out = jnp.where(lane < NRBF, f, 0.0)
    out = jnp.where(lane == NRBF, jnp.broadcast_to(rcut, out.shape), out)
    frc_ref[...] = out


def _filter_body(frc_ref, gxf_ref, fw1_ref, fb1_ref, fw2_ref, fb2_ref, xij_ref):
    f = frc_ref[...]
    t = jnp.dot(f, fw1_ref[...], preferred_element_type=jnp.float32) + fb1_ref[...]
    t = _ssp(t)
    w = jnp.dot(t, fw2_ref[...], preferred_element_type=jnp.float32) + fb2_ref[...]
    rcut = f[:, NRBF:NRBF + 1]
    xij_ref[...] = gxf_ref[...] * w * rcut


def _dense2_body(x_ref, ag0_ref, ag1_ref, gt0_ref, gt1_ref,
                 o1_ref, ob1_ref, o2_ref, ob2_ref, gb_ref, xn_ref):
    agg = ag0_ref[...] + ag1_ref[...]
    t = jnp.dot(agg, o1_ref[...], preferred_element_type=jnp.float32) + ob1_ref[...]
    t = _ssp(t)
    v = jnp.dot(t, o2_ref[...], preferred_element_type=jnp.float32) + ob2_ref[...]
    gat = (gt0_ref[...] + gt1_ref[...]) * (1.0 / HEADS)
    xn_ref[...] = x_ref[...] + v + gat + gb_ref[...]


def _dense1(xp, gw, i2f, As, Ad):
    grid = (NP_ // BN,)
    return pl.pallas_call(
        _dense1_body,
        grid=grid,
        in_specs=[_rows(BN, D), _full((D, HEADS * D)), _full((D, D)),
                  _full((D, 16)), _full((D, 16))],
        out_specs=[_rows(BN, HEADS * D), _rows(BN, D), _rows(BN, 16), _rows(BN, 16)],
        out_shape=[jax.ShapeDtypeStruct((NP_, HEADS * D), jnp.float32),
                   jax.ShapeDtypeStruct((NP_, D), jnp.float32),
                   jax.ShapeDtypeStruct((NP_, 16), jnp.float32),
                   jax.ShapeDtypeStruct((NP_, 16), jnp.float32)],
    )(xp, gw, i2f, As, Ad)


def _attnw(gs, gd):
    rows = gs.shape[0]
    return pl.pallas_call(
        _attnw_body,
        grid=(rows // BE,),
        in_specs=[_rows(BE, 16), _rows(BE, 16)],
        out_specs=_rows(BE, 16),
        out_shape=jax.ShapeDtypeStruct((rows, 16), jnp.float32),
    )(gs, gd)


def _edgea(w, gden):
    rows = w.shape[0]
    return pl.pallas_call(
        _edgea_body,
        grid=(rows // BE,),
        in_specs=[_rows(BE, 16), _rows(BE, 16)],
        out_specs=_rows(BE, 16),
        out_shape=jax.ShapeDtypeStruct((rows, 16), jnp.float32),
    )(w, gden)


def _contrib(a, gh):
    rows = a.shape[0]
    return pl.pallas_call(
        _contrib_body,
        grid=(rows // BE,),
        in_specs=[_rows(BE, 16), _rows(BE, HEADS * D)],
        out_specs=_rows(BE, D),
        out_shape=jax.ShapeDtypeStruct((rows, D), jnp.float32),
    )(a, gh)


def _frc(rp):
    rows = rp.shape[0]
    return pl.pallas_call(
        _frc_body,
        grid=(rows // BE,),
        in_specs=[_rows(BE, D)],
        out_specs=_rows(BE, D),
        out_shape=jax.ShapeDtypeStruct((rows, D), jnp.float32),
    )(rp)


def _filter(frc, gxf, fw1p, fb1, fw2, fb2):
    rows = frc.shape[0]
    return pl.pallas_call(
        _filter_body,
        grid=(rows // BE,),
        in_specs=[_rows(BE, D), _rows(BE, D), _full((D, D)), _full((1, D)),
                  _full((D, D)), _full((1, D))],
        out_specs=_rows(BE, D),
        out_shape=jax.ShapeDtypeStruct((rows, D), jnp.float32),
    )(frc, gxf, fw1p, fb1, fw2, fb2)


def _dense2(xp, ag0, ag1, gt0, gt1, o1, ob1, o2, ob2, gb):
    return pl.pallas_call(
        _dense2_body,
        grid=(NP_ // BN,),
        in_specs=[_rows(BN, D)] * 5 + [_full((D, D)), _full((1, D)),
                                       _full((D, D)), _full((1, D)), _full((1, D))],
        out_specs=_rows(BN, D),
        out_shape=jax.ShapeDtypeStruct((NP_, D), jnp.float32),
    )(xp, ag0, ag1, gt0, gt1, o1, ob1, o2, ob2, gb)


def _gather(table, idx):
    return jnp.take(table, idx, axis=0)


def _segsum(rows, idx, nseg):
    s = jax.ops.segment_sum(rows, idx, num_segments=nseg)
    return s, jnp.zeros_like(s)


def kernel(Z, Rij, idx_i, idx_j, emb, in2f_W, fw1, fb1, fw2, fb2,
           o1, ob1, o2, ob2, gW, ga_src, ga_dst, gb):
    f32 = jnp.float32
    loop = jnp.arange(N, dtype=idx_i.dtype)
    src2 = jnp.concatenate([idx_i, loop])
    dst2 = jnp.concatenate([idx_j, loop])

    gWr = gW.reshape(NINT, D, HEADS, D)
    As = jnp.einsum('tkhd,thd->tkh', gWr, ga_src)
    Ad = jnp.einsum('tkhd,thd->tkh', gWr, ga_dst)
    As = jnp.pad(As, ((0, 0), (0, 0), (0, 16 - HEADS)))
    Ad = jnp.pad(Ad, ((0, 0), (0, 0), (0, 16 - HEADS)))
    fw1p = jnp.pad(fw1, ((0, 0), (0, D - NRBF), (0, 0)))

    rp = jnp.pad(Rij, ((0, EP - E), (0, D - 3)))
    frc = _frc(rp)

    x = _gather(emb, Z)
    xp = jnp.pad(x, ((0, NP_ - N), (0, 0)))

    for t in range(NINT):
        h, xf, sa, sd = _dense1(xp, gW[t], in2f_W[t], As[t], Ad[t])
        gs = _gather(sa, src2)
        gd = _gather(sd, dst2)
        gs = jnp.pad(gs, ((0, E2P - E2), (0, 0)))
        gd = jnp.pad(gd, ((0, E2P - E2), (0, 0)))
        w = _attnw(gs, gd)[:E2]
        den0, den1 = _segsum(w, dst2, N)
        gden = _gather(den0 + den1, dst2)
        a = _edgea(jnp.pad(w, ((0, E2P - E2), (0, 0))),
                   jnp.pad(gden, ((0, E2P - E2), (0, 0)), constant_values=1.0))
        gh = _gather(h[:N], src2)
        gh = jnp.pad(gh, ((0, E2P - E2), (0, 0)))
        c = _contrib(a, gh)[:E2]
        gat0, gat1 = _segsum(c, dst2, N)
        gxf = _gather(xf[:N], idx_j)
        gxf = jnp.pad(gxf, ((0, EP - E), (0, 0)))
        xij = _filter(frc, gxf, fw1p[t], fb1[t][None, :], fw2[t], fb2[t][None, :])[:E]
        agg0, agg1 = _segsum(xij, idx_i, N)
        xp = _dense2(xp,
                     jnp.pad(agg0, ((0, NP_ - N), (0, 0))),
                     jnp.pad(agg1, ((0, NP_ - N), (0, 0))),
                     jnp.pad(gat0, ((0, NP_ - N), (0, 0))),
                     jnp.pad(gat1, ((0, NP_ - N), (0, 0))),
                     o1[t], ob1[t][None, :], o2[t], ob2[t][None, :],
                     gb[t][None, :])
    return xp[:N]


# speedup vs baseline: 7.3792x; 1.8985x over previous
import functools
import jax
from jax import lax
import jax.numpy as jnp
from jax.experimental import pallas as pl
from jax.experimental.pallas import tpu as pltpu
from jax.experimental.pallas import tpu_sc as plsc

N = 10000
E = 160000
D = 128
NRBF = 20
CUTOFF = 5.0
NINT = 3
HEADS = 10
E2 = E + N

BN = 512
BE = 512

NTILES = 32
NP_ = 10240
EG2 = 172032
EG = 163840
NG = 12288

_DELTA = CUTOFF / (NRBF - 1)
_COEFF = -0.5 / _DELTA ** 2


def _ssp(x):
    return jnp.logaddexp(x, 0.0) - jnp.log(2.0)


def _full(shape):
    return pl.BlockSpec(shape, lambda i: (0,) * len(shape))


def _rows(bs, width):
    return pl.BlockSpec((bs, width), lambda i: (i, 0))


def _dense1_body(x_ref, gw_ref, i2f_ref, as_ref, ad_ref,
                 h_ref, xf_ref, sa_ref, sd_ref):
    xb = x_ref[...]
    h_ref[...] = jnp.dot(xb, gw_ref[...], preferred_element_type=jnp.float32)
    xf_ref[...] = jnp.dot(xb, i2f_ref[...], preferred_element_type=jnp.float32)
    sa_ref[...] = jnp.dot(xb, as_ref[...], preferred_element_type=jnp.float32)
    sd_ref[...] = jnp.dot(xb, ad_ref[...], preferred_element_type=jnp.float32)


def _attnw_body(gs_ref, gd_ref, w_ref):
    e = gs_ref[...] + gd_ref[...]
    e = jnp.where(e > 0, e, 0.2 * e)
    w = jnp.exp(e)
    row = pl.program_id(0) * BE + lax.broadcasted_iota(jnp.int32, w.shape, 0)
    w_ref[...] = jnp.where(row < E2, w, 0.0)


def _edgea_body(w_ref, gd0_ref, gd1_ref, a_ref):
    a_ref[...] = w_ref[...] / (gd0_ref[...] + gd1_ref[...] + 1e-16)


def _contrib_body(a_ref, gh_ref, c_ref):
    a = a_ref[...]
    g = gh_ref[...]
    acc = g[:, 0:D] * a[:, 0:1]
    for hd in range(1, HEADS):
        acc += g[:, hd * D:(hd + 1) * D] * a[:, hd:hd + 1]
    c_ref[...] = acc


def _frc_body(r_ref, frc_ref):
    r = r_ref[...]
    d2 = jnp.sum(r * r, axis=1, keepdims=True)
    d = jnp.sqrt(d2)
    lane = jax.lax.broadcasted_iota(jnp.int32, r.shape, 1)
    offs = lane.astype(jnp.float32) * _DELTA
    f = jnp.exp(_COEFF * (d - offs) ** 2)
    rcut = 0.5 * (jnp.cos(jnp.pi * d / CUTOFF) + 1.0)
    rcut = rcut * (d < CUTOFF).astype(jnp.float32)
    out = jnp.where(lane < NRBF, f, 0.0)
    out = jnp.where(lane == NRBF, jnp.broadcast_to(rcut, out.shape), out)
    frc_ref[...] = out


def _filter_body(frc_ref, gxf_ref, fw1_ref, fb1_ref, fw2_ref, fb2_ref, xij_ref):
    f = frc_ref[...]
    t = jnp.dot(f, fw1_ref[...], preferred_element_type=jnp.float32) + fb1_ref[...]
    t = _ssp(t)
    w = jnp.dot(t, fw2_ref[...], preferred_element_type=jnp.float32) + fb2_ref[...]
    rcut = f[:, NRBF:NRBF + 1]
    xij = gxf_ref[...] * w * rcut
    row = pl.program_id(0) * BE + lax.broadcasted_iota(jnp.int32, xij.shape, 0)
    xij_ref[...] = jnp.where(row < E, xij, 0.0)


def _dense2_body(x_ref, ag0_ref, ag1_ref, gt0_ref, gt1_ref,
                 o1_ref, ob1_ref, o2_ref, ob2_ref, gb_ref, xn_ref):
    agg = ag0_ref[...] + ag1_ref[...]
    t = jnp.dot(agg, o1_ref[...], preferred_element_type=jnp.float32) + ob1_ref[...]
    t = _ssp(t)
    v = jnp.dot(t, o2_ref[...], preferred_element_type=jnp.float32) + ob2_ref[...]
    gat = (gt0_ref[...] + gt1_ref[...]) * (1.0 / HEADS)
    xn_ref[...] = x_ref[...] + v + gat + gb_ref[...]


def _dense1(xp, gw, i2f, As, Ad):
    grid = (NP_ // BN,)
    return pl.pallas_call(
        _dense1_body,
        grid=grid,
        in_specs=[_rows(BN, D), _full((D, HEADS * D)), _full((D, D)),
                  _full((D, D)), _full((D, D))],
        out_specs=[_rows(BN, HEADS * D), _rows(BN, D), _rows(BN, D), _rows(BN, D)],
        out_shape=[jax.ShapeDtypeStruct((NP_, HEADS * D), jnp.float32),
                   jax.ShapeDtypeStruct((NP_, D), jnp.float32),
                   jax.ShapeDtypeStruct((NP_, D), jnp.float32),
                   jax.ShapeDtypeStruct((NP_, D), jnp.float32)],
    )(xp, gw, i2f, As, Ad)


def _attnw(gs, gd):
    rows = gs.shape[0]
    return pl.pallas_call(
        _attnw_body,
        grid=(rows // BE,),
        in_specs=[_rows(BE, D), _rows(BE, D)],
        out_specs=_rows(BE, D),
        out_shape=jax.ShapeDtypeStruct((rows, D), jnp.float32),
    )(gs, gd)


def _edgea(w, gd0, gd1):
    rows = w.shape[0]
    return pl.pallas_call(
        _edgea_body,
        grid=(rows // BE,),
        in_specs=[_rows(BE, D), _rows(BE, D), _rows(BE, D)],
        out_specs=_rows(BE, D),
        out_shape=jax.ShapeDtypeStruct((rows, D), jnp.float32),
    )(w, gd0, gd1)


def _contrib(a, gh):
    rows = a.shape[0]
    return pl.pallas_call(
        _contrib_body,
        grid=(rows // BE,),
        in_specs=[_rows(BE, D), _rows(BE, HEADS * D)],
        out_specs=_rows(BE, D),
        out_shape=jax.ShapeDtypeStruct((rows, D), jnp.float32),
    )(a, gh)


def _frc(rp):
    rows = rp.shape[0]
    return pl.pallas_call(
        _frc_body,
        grid=(rows // BE,),
        in_specs=[_rows(BE, D)],
        out_specs=_rows(BE, D),
        out_shape=jax.ShapeDtypeStruct((rows, D), jnp.float32),
    )(rp)


def _filter(frc, gxf, fw1p, fb1, fw2, fb2):
    rows = frc.shape[0]
    return pl.pallas_call(
        _filter_body,
        grid=(rows // BE,),
        in_specs=[_rows(BE, D), _rows(BE, D), _full((D, D)), _full((1, D)),
                  _full((D, D)), _full((1, D))],
        out_specs=_rows(BE, D),
        out_shape=jax.ShapeDtypeStruct((rows, D), jnp.float32),
    )(frc, gxf, fw1p, fb1, fw2, fb2)


def _dense2(xp, ag0, ag1, gt0, gt1, o1, ob1, o2, ob2, gb):
    return pl.pallas_call(
        _dense2_body,
        grid=(NP_ // BN,),
        in_specs=[_rows(BN, D)] * 5 + [_full((D, D)), _full((1, D)),
                                       _full((D, D)), _full((1, D)), _full((1, D))],
        out_specs=_rows(BN, D),
        out_shape=jax.ShapeDtypeStruct((NP_, D), jnp.float32),
    )(xp, ag0, ag1, gt0, gt1, o1, ob1, o2, ob2, gb)


_SC_MESH = plsc.VectorSubcoreMesh(core_axis_name="c", subcore_axis_name="s")


def _sc_gather(table, idx, chunk):
    K = idx.shape[0]
    width = table.shape[1]
    per_tile = K // NTILES
    nsteps = per_tile // chunk

    @functools.partial(
        pl.kernel, mesh=_SC_MESH,
        out_type=jax.ShapeDtypeStruct((K, width), jnp.float32),
        scratch_types=[pltpu.VMEM((chunk,), jnp.int32),
                       pltpu.VMEM((chunk, width), jnp.float32),
                       pltpu.SemaphoreType.DMA],
    )
    def k(table_hbm, idx_hbm, out_hbm, idx_v, rows_v, sem):
        wid = lax.axis_index("s") * 2 + lax.axis_index("c")
        base = wid * per_tile

        @pl.loop(0, nsteps)
        def _(i):
            off = base + i * chunk
            pltpu.sync_copy(idx_hbm.at[pl.ds(off, chunk)], idx_v)
            pltpu.async_copy(table_hbm.at[idx_v], rows_v, sem).wait()
            pltpu.sync_copy(rows_v, out_hbm.at[pl.ds(off, chunk)])

    return k(table, idx)


def _sc_scatter_add(rows, idx, zeros, chunk):
    K, width = rows.shape
    n0 = zeros.shape[0]
    per_tile = K // NTILES
    nsteps = per_tile // chunk
    zrows = n0 // 16

    @functools.partial(
        pl.kernel, mesh=_SC_MESH,
        out_type=jax.ShapeDtypeStruct((2, n0, width), jnp.float32),
        scratch_types=[pltpu.VMEM((chunk,), jnp.int32),
                       pltpu.VMEM((chunk, width), jnp.float32),
                       pltpu.VMEM_SHARED((n0, width), jnp.float32),
                       pltpu.SemaphoreType.DMA],
    )
    def k(rows_hbm, idx_hbm, zeros_hbm, out_hbm, idx_v, rows_v, acc_sh, sem):
        c = lax.axis_index("c")
        s = lax.axis_index("s")
        wid = s * 2 + c
        pltpu.sync_copy(zeros_hbm.at[pl.ds(s * zrows, zrows)],
                        acc_sh.at[pl.ds(s * zrows, zrows)])
        plsc.subcore_barrier()
        base = wid * per_tile

        @pl.loop(0, nsteps)
        def _(i):
            off = base + i * chunk
            pltpu.sync_copy(idx_hbm.at[pl.ds(off, chunk)], idx_v)
            pltpu.sync_copy(rows_hbm.at[pl.ds(off, chunk)], rows_v)
            pltpu.sync_copy(rows_v, acc_sh.at[idx_v], add=True)

        plsc.subcore_barrier()
        pltpu.sync_copy(acc_sh.at[pl.ds(s * zrows, zrows)],
                        out_hbm.at[c].at[pl.ds(s * zrows, zrows)])

    return k(rows, idx, zeros)


def kernel(Z, Rij, idx_i, idx_j, emb, in2f_W, fw1, fb1, fw2, fb2,
           o1, ob1, o2, ob2, gW, ga_src, ga_dst, gb):
    i32 = jnp.int32
    loop = jnp.arange(N, dtype=i32)
    src2 = jnp.concatenate([idx_i.astype(i32), loop])
    dst2 = jnp.concatenate([idx_j.astype(i32), loop])
    src2p = jnp.pad(src2, (0, EG2 - E2))
    dst2p = jnp.pad(dst2, (0, EG2 - E2))
    idx_jp = jnp.pad(idx_j.astype(i32), (0, EG - E))
    idx_ip = jnp.pad(idx_i.astype(i32), (0, EG - E))
    zp = jnp.pad(Z.astype(i32), (0, NG - N))
    zeros128 = jnp.zeros((NP_, D), jnp.float32)

    gWr = gW.reshape(NINT, D, HEADS, D)
    As = jnp.einsum('tkhd,thd->tkh', gWr, ga_src)
    Ad = jnp.einsum('tkhd,thd->tkh', gWr, ga_dst)
    As = jnp.pad(As, ((0, 0), (0, 0), (0, D - HEADS)))
    Ad = jnp.pad(Ad, ((0, 0), (0, 0), (0, D - HEADS)))
    fw1p = jnp.pad(fw1, ((0, 0), (0, D - NRBF), (0, 0)))

    rp = jnp.pad(Rij, ((0, EG - E), (0, D - 3)))
    frc = _frc(rp)

    xp = _sc_gather(emb, zp, 128)[:NP_]

    for t in range(NINT):
        h, xf, sa, sd = _dense1(xp, gW[t], in2f_W[t], As[t], Ad[t])
        gs = _sc_gather(sa, src2p, 128)
        gd = _sc_gather(sd, dst2p, 128)
        w = _attnw(gs, gd)
        den = _sc_scatter_add(w, dst2p, zeros128, 128)
        gd0 = _sc_gather(den[0], dst2p, 128)
        gd1 = _sc_gather(den[1], dst2p, 128)
        a = _edgea(w, gd0, gd1)
        gh = _sc_gather(h, src2p, 32)
        c = _contrib(a, gh)
        gat = _sc_scatter_add(c, dst2p, zeros128, 128)
        gxf = _sc_gather(xf, idx_jp, 128)
        xij = _filter(frc, gxf, fw1p[t], fb1[t][None, :], fw2[t], fb2[t][None, :])
        agg = _sc_scatter_add(xij, idx_ip, zeros128, 128)
        xp = _dense2(xp,
                     agg[0], agg[1], gat[0], gat[1],
                     o1[t], ob1[t][None, :], o2[t], ob2[t][None, :],
                     gb[t][None, :])
    return xp[:N]


# speedup vs baseline: 10.1317x; 1.3730x over previous
import functools
import jax
from jax import lax
import jax.numpy as jnp
from jax.experimental import pallas as pl
from jax.experimental.pallas import tpu as pltpu
from jax.experimental.pallas import tpu_sc as plsc

N = 10000
E = 160000
D = 128
NRBF = 20
CUTOFF = 5.0
NINT = 3
HEADS = 10
E2 = E + N

BN = 512
BE = 512

NTILES = 32
NP_ = 10240
EG2 = 172032
EG = 163840
NG = 12288

_DELTA = CUTOFF / (NRBF - 1)
_COEFF = -0.5 / _DELTA ** 2


def _ssp(x):
    return jnp.logaddexp(x, 0.0) - jnp.log(2.0)


def _full(shape):
    return pl.BlockSpec(shape, lambda i: (0,) * len(shape))


def _rows(bs, width):
    return pl.BlockSpec((bs, width), lambda i: (i, 0))


def _rne_bf16_bits(f):
    bits = jax.lax.bitcast_convert_type(f, jnp.int32)
    lsb = jax.lax.shift_right_logical(bits, 16) & 1
    r = jax.lax.shift_right_logical(bits + 0x7FFF + lsb, 16)
    return r & 0xFFFF


def _pack_bf16_pairs(h):
    parts = []
    for hd in range(HEADS):
        a = h[:, hd * D:hd * D + 64]
        b = h[:, hd * D + 64:hd * D + D]
        parts.append(_rne_bf16_bits(a) |
                     jax.lax.shift_left(_rne_bf16_bits(b), 16))
    return jnp.concatenate(parts, axis=1)


def _dense1_body(x_ref, gw_ref, i2f_ref, as_ref, ad_ref,
                 h_ref, xf_ref, sa_ref, sd_ref):
    xb = x_ref[...]
    h = jnp.dot(xb, gw_ref[...], preferred_element_type=jnp.float32)
    h_ref[...] = _pack_bf16_pairs(h)
    xf_ref[...] = jnp.dot(xb, i2f_ref[...], preferred_element_type=jnp.float32)
    sa_ref[...] = jnp.dot(xb, as_ref[...], preferred_element_type=jnp.float32)
    sd_ref[...] = jnp.dot(xb, ad_ref[...], preferred_element_type=jnp.float32)


def _attnw_body(gs_ref, gd_ref, w_ref):
    e = gs_ref[...] + gd_ref[...]
    e = jnp.where(e > 0, e, 0.2 * e)
    w = jnp.exp(e)
    row = pl.program_id(0) * BE + lax.broadcasted_iota(jnp.int32, w.shape, 0)
    w_ref[...] = jnp.where(row < E2, w, 0.0)


def _contrib_body(w_ref, gden_ref, gh_ref, c_ref):
    a = w_ref[...] / (gden_ref[...] + 1e-16)
    g = gh_ref[...]
    acc_lo = jnp.zeros((a.shape[0], 64), jnp.float32)
    acc_hi = jnp.zeros((a.shape[0], 64), jnp.float32)
    for hd in range(HEADS):
        pk = g[:, hd * 64:(hd + 1) * 64]
        lo = jax.lax.bitcast_convert_type(
            jax.lax.shift_left(pk, 16), jnp.float32)
        hi = jax.lax.bitcast_convert_type(
            pk & jnp.int32(-65536), jnp.float32)
        ah = a[:, hd:hd + 1]
        acc_lo = acc_lo + lo * ah
        acc_hi = acc_hi + hi * ah
    c_ref[...] = jnp.concatenate([acc_lo, acc_hi], axis=1)


def _frc_body(r_ref, frc_ref):
    r = r_ref[...]
    d2 = jnp.sum(r * r, axis=1, keepdims=True)
    d = jnp.sqrt(d2)
    lane = jax.lax.broadcasted_iota(jnp.int32, r.shape, 1)
    offs = lane.astype(jnp.float32) * _DELTA
    f = jnp.exp(_COEFF * (d - offs) ** 2)
    rcut = 0.5 * (jnp.cos(jnp.pi * d / CUTOFF) + 1.0)
    rcut = rcut * (d < CUTOFF).astype(jnp.float32)
    out = jnp.where(lane < NRBF, f, 0.0)
    out = jnp.where(lane == NRBF, jnp.broadcast_to(rcut, out.shape), out)
    frc_ref[...] = out


def _filter_body(frc_ref, gxf_ref, fw1_ref, fb1_ref, fw2_ref, fb2_ref, xij_ref):
    f = frc_ref[...]
    t = jnp.dot(f, fw1_ref[...], preferred_element_type=jnp.float32) + fb1_ref[...]
    t = _ssp(t)
    w = jnp.dot(t, fw2_ref[...], preferred_element_type=jnp.float32) + fb2_ref[...]
    rcut = f[:, NRBF:NRBF + 1]
    xij = gxf_ref[...] * w * rcut
    row = pl.program_id(0) * BE + lax.broadcasted_iota(jnp.int32, xij.shape, 0)
    xij_ref[...] = jnp.where(row < E, xij, 0.0)


def _dense2_body(x_ref, ag0_ref, ag1_ref, gt0_ref, gt1_ref,
                 o1_ref, ob1_ref, o2_ref, ob2_ref, gb_ref, xn_ref):
    agg = ag0_ref[...] + ag1_ref[...]
    t = jnp.dot(agg, o1_ref[...], preferred_element_type=jnp.float32) + ob1_ref[...]
    t = _ssp(t)
    v = jnp.dot(t, o2_ref[...], preferred_element_type=jnp.float32) + ob2_ref[...]
    gat = (gt0_ref[...] + gt1_ref[...]) * (1.0 / HEADS)
    xn_ref[...] = x_ref[...] + v + gat + gb_ref[...]


def _dense1(xp, gw, i2f, As, Ad):
    grid = (NP_ // BN,)
    return pl.pallas_call(
        _dense1_body,
        grid=grid,
        in_specs=[_rows(BN, D), _full((D, HEADS * D)), _full((D, D)),
                  _full((D, D)), _full((D, D))],
        out_specs=[_rows(BN, HEADS * 64), _rows(BN, D), _rows(BN, D), _rows(BN, D)],
        out_shape=[jax.ShapeDtypeStruct((NP_, HEADS * 64), jnp.int32),
                   jax.ShapeDtypeStruct((NP_, D), jnp.float32),
                   jax.ShapeDtypeStruct((NP_, D), jnp.float32),
                   jax.ShapeDtypeStruct((NP_, D), jnp.float32)],
    )(xp, gw, i2f, As, Ad)


def _attnw(gs, gd):
    rows = gs.shape[0]
    return pl.pallas_call(
        _attnw_body,
        grid=(rows // BE,),
        in_specs=[_rows(BE, D), _rows(BE, D)],
        out_specs=_rows(BE, D),
        out_shape=jax.ShapeDtypeStruct((rows, D), jnp.float32),
    )(gs, gd)


def _contrib(w, gden, gh):
    rows = w.shape[0]
    return pl.pallas_call(
        _contrib_body,
        grid=(rows // BE,),
        in_specs=[_rows(BE, D), _rows(BE, D), _rows(BE, HEADS * 64)],
        out_specs=_rows(BE, D),
        out_shape=jax.ShapeDtypeStruct((rows, D), jnp.float32),
    )(w, gden, gh)


def _frc(rp):
    rows = rp.shape[0]
    return pl.pallas_call(
        _frc_body,
        grid=(rows // BE,),
        in_specs=[_rows(BE, D)],
        out_specs=_rows(BE, D),
        out_shape=jax.ShapeDtypeStruct((rows, D), jnp.float32),
    )(rp)


def _filter(frc, gxf, fw1p, fb1, fw2, fb2):
    rows = frc.shape[0]
    return pl.pallas_call(
        _filter_body,
        grid=(rows // BE,),
        in_specs=[_rows(BE, D), _rows(BE, D), _full((D, D)), _full((1, D)),
                  _full((D, D)), _full((1, D))],
        out_specs=_rows(BE, D),
        out_shape=jax.ShapeDtypeStruct((rows, D), jnp.float32),
    )(frc, gxf, fw1p, fb1, fw2, fb2)


def _addp_body(a_ref, b_ref, o_ref):
    o_ref[...] = a_ref[...] + b_ref[...]


def _addp(a, b):
    rows = a.shape[0]
    return pl.pallas_call(
        _addp_body,
        grid=(rows // BN,),
        in_specs=[_rows(BN, D), _rows(BN, D)],
        out_specs=_rows(BN, D),
        out_shape=jax.ShapeDtypeStruct((rows, D), jnp.float32),
    )(a, b)


def _dense2(xp, ag0, ag1, gt0, gt1, o1, ob1, o2, ob2, gb):
    return pl.pallas_call(
        _dense2_body,
        grid=(NP_ // BN,),
        in_specs=[_rows(BN, D)] * 5 + [_full((D, D)), _full((1, D)),
                                       _full((D, D)), _full((1, D)), _full((1, D))],
        out_specs=_rows(BN, D),
        out_shape=jax.ShapeDtypeStruct((NP_, D), jnp.float32),
    )(xp, ag0, ag1, gt0, gt1, o1, ob1, o2, ob2, gb)


_SC_MESH = plsc.VectorSubcoreMesh(core_axis_name="c", subcore_axis_name="s")


def _sc_gather(table, idx, chunk, nb):
    K = idx.shape[0]
    width = table.shape[1]
    dt = table.dtype
    per_tile = K // NTILES
    nsteps = per_tile // chunk
    ngroups = -(-nsteps // nb)
    gchunk = nb * chunk

    @functools.partial(
        pl.kernel, mesh=_SC_MESH,
        out_type=jax.ShapeDtypeStruct((K, width), dt),
        scratch_types=[pltpu.VMEM((per_tile,), jnp.int32),
                       pltpu.VMEM((gchunk, width), dt),
                       pltpu.VMEM((gchunk, width), dt),
                       pltpu.SemaphoreType.DMA,
                       pltpu.SemaphoreType.DMA,
                       pltpu.SemaphoreType.DMA],
    )
    def k(table_hbm, idx_hbm, out_hbm, idx_v, buf_a, buf_b, sem_g, sem_wa, sem_wb):
        wid = lax.axis_index("s") * 2 + lax.axis_index("c")
        base = wid * per_tile
        pltpu.sync_copy(idx_hbm.at[pl.ds(base, per_tile)], idx_v)

        def do_group(g, buf, sem_w):
            @pl.when(g >= 2)
            def _():
                pltpu.make_async_copy(buf, out_hbm.at[pl.ds(base, gchunk)],
                                      sem_w).wait()
            for b in range(nb):
                pltpu.async_copy(
                    table_hbm.at[idx_v.at[pl.ds((g * nb + b) * chunk, chunk)]],
                    buf.at[pl.ds(b * chunk, chunk)], sem_g)
            for b in range(nb):
                pltpu.make_async_copy(
                    table_hbm.at[idx_v.at[pl.ds((g * nb + b) * chunk, chunk)]],
                    buf.at[pl.ds(b * chunk, chunk)], sem_g).wait()
            pltpu.async_copy(buf, out_hbm.at[pl.ds(base + g * gchunk, gchunk)],
                             sem_w)

        @pl.loop(0, ngroups, step=2)
        def _(g):
            do_group(g, buf_a, sem_wa)

            @pl.when(g + 1 < ngroups)
            def _():
                do_group(g + 1, buf_b, sem_wb)

        pltpu.make_async_copy(buf_a, out_hbm.at[pl.ds(base, gchunk)],
                              sem_wa).wait()

        @pl.when(ngroups >= 2)
        def _():
            pltpu.make_async_copy(buf_b, out_hbm.at[pl.ds(base, gchunk)],
                                  sem_wb).wait()

    return k(table, idx)


def _sc_scatter_add(rows, idx, zeros, chunk=64):
    K, width = rows.shape
    n0 = zeros.shape[0]
    per_tile = K // NTILES
    nsteps = per_tile // chunk
    zrows = n0 // 16

    scr = ([pltpu.VMEM((chunk,), jnp.int32) for _ in range(3)] +
           [pltpu.VMEM((chunk, width), jnp.float32) for _ in range(3)] +
           [pltpu.VMEM_SHARED((n0, width), jnp.float32)] +
           [pltpu.SemaphoreType.DMA for _ in range(6)])

    @functools.partial(
        pl.kernel, mesh=_SC_MESH,
        out_type=jax.ShapeDtypeStruct((2, n0, width), jnp.float32),
        scratch_types=scr,
    )
    def k(rows_hbm, idx_hbm, zeros_hbm, out_hbm, *refs):
        ib = refs[0:3]
        rb = refs[3:6]
        acc_sh = refs[6]
        sem_l = refs[7:10]
        sem_s = refs[10:13]
        c = lax.axis_index("c")
        s = lax.axis_index("s")
        wid = s * 2 + c
        base = wid * per_tile
        pltpu.sync_copy(zeros_hbm.at[pl.ds(s * zrows, zrows)],
                        acc_sh.at[pl.ds(s * zrows, zrows)])
        plsc.subcore_barrier()

        def fire_load(i, b):
            off = base + i * chunk
            pltpu.async_copy(rows_hbm.at[pl.ds(off, chunk)], rb[b], sem_l[b])
            pltpu.async_copy(idx_hbm.at[pl.ds(off, chunk)], ib[b], sem_l[b])

        def drain_load(b):
            pltpu.make_async_copy(rows_hbm.at[pl.ds(base, chunk)], rb[b],
                                  sem_l[b]).wait()
            pltpu.make_async_copy(idx_hbm.at[pl.ds(base, chunk)], ib[b],
                                  sem_l[b]).wait()

        def drain_scatter(b):
            pltpu.make_async_copy(rb[b], acc_sh.at[ib[b]], sem_s[b]).wait()

        fire_load(0, 0)

        @pl.when(nsteps > 1)
        def _():
            fire_load(1, 1)

        @pl.loop(0, nsteps, step=3)
        def _(i0):
            for sub in range(3):
                i = i0 + sub
                b = sub

                @pl.when(i < nsteps)
                def _(i=i, b=b):
                    drain_load(b)
                    pltpu.async_copy(rb[b], acc_sh.at[ib[b]], sem_s[b],
                                     add=True)

                    @pl.when(i >= 1)
                    def _(b=b):
                        drain_scatter((b + 2) % 3)

                    @pl.when(i + 2 < nsteps)
                    def _(i=i, b=b):
                        fire_load(i + 2, (b + 2) % 3)

        drain_scatter((nsteps - 1) % 3)
        plsc.subcore_barrier()
        pltpu.sync_copy(acc_sh.at[pl.ds(s * zrows, zrows)],
                        out_hbm.at[c].at[pl.ds(s * zrows, zrows)])

    return k(rows, idx, zeros)


def kernel(Z, Rij, idx_i, idx_j, emb, in2f_W, fw1, fb1, fw2, fb2,
           o1, ob1, o2, ob2, gW, ga_src, ga_dst, gb):
    i32 = jnp.int32
    loop = jnp.arange(N, dtype=i32)
    src2 = jnp.concatenate([idx_i.astype(i32), loop])
    dst2 = jnp.concatenate([idx_j.astype(i32), loop])
    src2p = jnp.pad(src2, (0, EG2 - E2))
    dst2p = jnp.pad(dst2, (0, EG2 - E2))
    idx_jp = jnp.pad(idx_j.astype(i32), (0, EG - E))
    idx_ip = jnp.pad(idx_i.astype(i32), (0, EG - E))
    zp = jnp.pad(Z.astype(i32), (0, NG - N))
    zeros128 = jnp.zeros((NP_, D), jnp.float32)

    gWr = gW.reshape(NINT, D, HEADS, D)
    As = jnp.einsum('tkhd,thd->tkh', gWr, ga_src)
    Ad = jnp.einsum('tkhd,thd->tkh', gWr, ga_dst)
    As = jnp.pad(As, ((0, 0), (0, 0), (0, D - HEADS)))
    Ad = jnp.pad(Ad, ((0, 0), (0, 0), (0, D - HEADS)))
    fw1p = jnp.pad(fw1, ((0, 0), (0, D - NRBF), (0, 0)))

    rp = jnp.pad(Rij, ((0, EG - E), (0, D - 3)))
    frc = _frc(rp)

    xp = _sc_gather(emb, zp, 32, 2)[:NP_]

    for t in range(NINT):
        h, xf, sa, sd = _dense1(xp, gW[t], in2f_W[t], As[t], Ad[t])
        gs = _sc_gather(sa, src2p, 64, 4)
        gd = _sc_gather(sd, dst2p, 64, 4)
        w = _attnw(gs, gd)
        den = _sc_scatter_add(w, dst2p, zeros128)
        dsum = _addp(den[0], den[1])
        gden = _sc_gather(dsum, dst2p, 64, 4)
        gh = _sc_gather(h, src2p, 32, 3)
        c = _contrib(w, gden, gh)
        gat = _sc_scatter_add(c, dst2p, zeros128)
        gxf = _sc_gather(xf, idx_jp, 64, 4)
        xij = _filter(frc, gxf, fw1p[t], fb1[t][None, :], fw2[t], fb2[t][None, :])
        agg = _sc_scatter_add(xij, idx_ip, zeros128)
        xp = _dense2(xp,
                     agg[0], agg[1], gat[0], gat[1],
                     o1[t], ob1[t][None, :], o2[t], ob2[t][None, :],
                     gb[t][None, :])
    return xp[:N]
